# baseline (speedup 1.0000x reference)
import functools

import jax
import jax.numpy as jnp
from jax import lax
from jax.experimental import pallas as pl
from jax.experimental.pallas import tpu as pltpu
from jax.experimental.pallas import tpu_sc as plsc

T = 2048
D = 768
H = 2048
E = 8
KTOP = 2
EP = 128
TK = T * KTOP
TM = 128
NT = TK // TM
NI = NT + E - 1
TT = 2048
NW = 32
TPW = T // NW
WL = 16
WSL = 128


def _gating_body(x_ref, wg_ref, bg_ref, w_ref, lp_ref, ti_ref, wb1_ref,
                 wb2_ref):
    lg = jnp.dot(x_ref[...], wg_ref[...], preferred_element_type=jnp.float32)
    lg = lg + bg_ref[...]
    col = lax.broadcasted_iota(jnp.int32, (TT, EP), 1)
    m = jnp.max(lg, axis=1, keepdims=True)
    el = jnp.exp(lg - m)
    s = jnp.sum(el, axis=1, keepdims=True)
    w = el / s
    w_ref[...] = w
    lp_ref[...] = lg - m - jnp.log(s)
    wm = jnp.where(col < E, w, -1.0)
    w1 = jnp.max(wm, axis=1, keepdims=True)
    i1 = jnp.min(jnp.where(wm == w1, col, EP), axis=1, keepdims=True)
    wm2 = jnp.where(col == i1, -1.0, wm)
    w2 = jnp.max(wm2, axis=1, keepdims=True)
    i2 = jnp.min(jnp.where(wm2 == w2, col, EP), axis=1, keepdims=True)
    den = w1 + w2 + 1e-9
    wn1 = w1 / den
    wn2 = w2 / den
    ti_ref[...] = jnp.where(col == 0, i1, jnp.where(col == 1, i2, 0))
    wb1_ref[...] = jnp.broadcast_to(wn1, (TT, EP))
    wb2_ref[...] = jnp.broadcast_to(wn2, (TT, EP))


def _gating(xf, wgp, bgp):
    grid = (T // TT,)
    f32 = jnp.float32
    return pl.pallas_call(
        _gating_body,
        grid=grid,
        in_specs=[
            pl.BlockSpec((TT, D), lambda i: (i, 0)),
            pl.BlockSpec((D, EP), lambda i: (0, 0)),
            pl.BlockSpec((1, EP), lambda i: (0, 0)),
        ],
        out_specs=[
            pl.BlockSpec((TT, EP), lambda i: (i, 0)),
            pl.BlockSpec((TT, EP), lambda i: (i, 0)),
            pl.BlockSpec((TT, EP), lambda i: (i, 0)),
            pl.BlockSpec((TT, EP), lambda i: (i, 0)),
            pl.BlockSpec((TT, EP), lambda i: (i, 0)),
        ],
        out_shape=[
            jax.ShapeDtypeStruct((T, EP), f32),
            jax.ShapeDtypeStruct((T, EP), f32),
            jax.ShapeDtypeStruct((T, EP), jnp.int32),
            jax.ShapeDtypeStruct((T, EP), f32),
            jax.ShapeDtypeStruct((T, EP), f32),
        ],
    )(xf, wgp, bgp)


def _sc_mesh():
    return plsc.VectorSubcoreMesh(core_axis_name="c", subcore_axis_name="s",
                                  num_cores=2)


def _worker_id():
    return lax.axis_index("s") * 2 + lax.axis_index("c")


def _dispatch_body(x_hbm, pe_hbm, po_hbm, xs_hbm, pe_v, po_v, xv, sem):
    base = _worker_id() * TPW
    pltpu.sync_copy(pe_hbm.at[pl.ds(base, TPW)], pe_v)
    pltpu.sync_copy(po_hbm.at[pl.ds(base, TPW)], po_v)
    pltpu.sync_copy(x_hbm.at[pl.ds(base, TPW)], xv)
    pltpu.async_copy(xv, xs_hbm.at[pe_v], sem).wait()
    pltpu.async_copy(xv, xs_hbm.at[po_v], sem).wait()


def _dispatch(xf, pe, po):
    return pl.kernel(
        _dispatch_body,
        out_type=jax.ShapeDtypeStruct((TK, D), jnp.float32),
        mesh=_sc_mesh(),
        scratch_types=[
            pltpu.VMEM((TPW,), jnp.int32),
            pltpu.VMEM((TPW,), jnp.int32),
            pltpu.VMEM((TPW, D), jnp.float32),
            pltpu.SemaphoreType.DMA,
        ],
    )(xf, pe, po)


def _mlp_body(off, xs_ref, w1_ref, b1_ref, w2_ref, b2_ref, out_ref):
    e = pl.program_id(0)
    lo = off[e]
    hi = off[e + 1]

    @pl.when(e == 0)
    def _():
        out_ref[...] = jnp.zeros_like(out_ref)

    t0 = lo // TM
    nt = jnp.where(hi > lo, (hi - 1) // TM - t0 + 1, 0)

    def tile(j, _):
        t = t0 + j
        r = pl.ds(t * TM, TM)
        h = jnp.dot(xs_ref[r, :], w1_ref[0],
                    preferred_element_type=jnp.float32)
        h = h + b1_ref[0]
        h = 0.5 * h * (1.0 + lax.erf(h * 0.7071067811865476))
        o = jnp.dot(h, w2_ref[0], preferred_element_type=jnp.float32)
        o = o + b2_ref[0]
        rows = t * TM + lax.broadcasted_iota(jnp.int32, (TM, 1), 0)
        keep = (rows >= lo) & (rows < hi)
        o = jnp.where(keep, o, 0.0)
        out_ref[r, :] += o
        return 0

    lax.fori_loop(0, nt, tile, 0)


def _grouped_mlp(xs, W1, b1, W2, b2, off9):
    grid_spec = pltpu.PrefetchScalarGridSpec(
        num_scalar_prefetch=1,
        grid=(E,),
        in_specs=[
            pl.BlockSpec((TK, D), lambda e, off: (0, 0)),
            pl.BlockSpec((1, D, H), lambda e, off: (e, 0, 0)),
            pl.BlockSpec((1, 1, H), lambda e, off: (e, 0, 0)),
            pl.BlockSpec((1, H, D), lambda e, off: (e, 0, 0)),
            pl.BlockSpec((1, 1, D), lambda e, off: (e, 0, 0)),
        ],
        out_specs=pl.BlockSpec((TK, D), lambda e, off: (0, 0)),
    )
    return pl.pallas_call(
        _mlp_body,
        grid_spec=grid_spec,
        out_shape=jax.ShapeDtypeStruct((TK, D), jnp.float32),
    )(off9, xs, W1,
      b1.reshape(E, 1, H), W2, b2.reshape(E, 1, D))


def _combine_body(ys_hbm, pe_hbm, po_hbm, wb1_hbm, wb2_hbm, y_hbm,
                  pe_v, po_v, w1_v, w2_v, av, bv, sem):
    base = _worker_id() * TPW
    pltpu.sync_copy(pe_hbm.at[pl.ds(base, TPW)], pe_v)
    pltpu.sync_copy(po_hbm.at[pl.ds(base, TPW)], po_v)
    pltpu.sync_copy(wb1_hbm.at[pl.ds(base, TPW)], w1_v)
    pltpu.sync_copy(wb2_hbm.at[pl.ds(base, TPW)], w2_v)
    pltpu.async_copy(ys_hbm.at[pe_v], av, sem).wait()
    pltpu.async_copy(ys_hbm.at[po_v], bv, sem).wait()

    @plsc.parallel_loop(0, TPW * (D // WL), unroll=8)
    def _fma(i):
        t = i & (TPW - 1)
        j = lax.shift_right_logical(i, 6)
        sl = pl.ds(j * WL, WL)
        wsl = pl.ds(0, WL)
        av[t, sl] = av[t, sl] * w1_v[t, wsl] + bv[t, sl] * w2_v[t, wsl]

    pltpu.sync_copy(av, y_hbm.at[pl.ds(base, TPW)])


def _combine(ys, pe, po, wb1, wb2):
    return pl.kernel(
        _combine_body,
        out_type=jax.ShapeDtypeStruct((T, D), jnp.float32),
        mesh=_sc_mesh(),
        scratch_types=[
            pltpu.VMEM((TPW,), jnp.int32),
            pltpu.VMEM((TPW,), jnp.int32),
            pltpu.VMEM((TPW, EP), jnp.float32),
            pltpu.VMEM((TPW, EP), jnp.float32),
            pltpu.VMEM((TPW, D), jnp.float32),
            pltpu.VMEM((TPW, D), jnp.float32),
            pltpu.SemaphoreType.DMA,
        ],
    )(ys, pe, po, wb1, wb2)


def _schedule(ti):
    ex = ti.reshape(-1)
    onehot = (ex[:, None] == jnp.arange(E)[None, :]).astype(jnp.int32)
    counts = jnp.sum(onehot, axis=0)
    off9 = jnp.concatenate([jnp.zeros((1,), jnp.int32),
                            jnp.cumsum(counts).astype(jnp.int32)])
    rank = jnp.cumsum(onehot, axis=0) - onehot
    rank_p = jnp.take_along_axis(rank, ex[:, None], axis=1)[:, 0]
    pos = off9[ex] + rank_p
    pe = pos[0::2].astype(jnp.int32)
    po = pos[1::2].astype(jnp.int32)
    return pe, po, off9


def kernel(x, Wg, bg, W1, b1, W2, b2):
    b, s, d = x.shape
    xf = x.reshape(T, D)
    wgp = jnp.pad(Wg, ((0, 0), (0, EP - E)))
    bgp = jnp.pad(bg, (0, EP - E), constant_values=-1e30).reshape(1, EP)

    w_pad, lp_pad, ti_pad, wb1, wb2 = _gating(xf, wgp, bgp)
    weights = w_pad[:, :E]
    log_probs = lp_pad[:, :E]
    ti = ti_pad[:, :KTOP]

    pe, po, off9 = _schedule(ti)
    xs = _dispatch(xf, pe, po)
    ys = _grouped_mlp(xs, W1, b1, W2, b2, off9)
    y = _combine(ys, pe, po, wb1, wb2)

    return (y.reshape(b, s, d), log_probs.reshape(b, s, E),
            weights.reshape(b, s, E), ti.reshape(b, s, KTOP))


# speedup vs baseline: 1.0426x; 1.0426x over previous
import functools

import jax
import jax.numpy as jnp
from jax import lax
from jax.experimental import pallas as pl
from jax.experimental.pallas import tpu as pltpu
from jax.experimental.pallas import tpu_sc as plsc

T = 2048
D = 768
H = 2048
E = 8
KTOP = 2
EP = 128
TK = T * KTOP
TM = 256
NT = TK // TM
NI = NT + E - 1
TT = 2048
NW = 32
TPW = T // NW
WL = 16
WSL = 128


def _gating_body(x_ref, wg_ref, bg_ref, w_ref, lp_ref, ti_ref, wb1_ref,
                 wb2_ref):
    lg = jnp.dot(x_ref[...], wg_ref[...], preferred_element_type=jnp.float32)
    lg = lg + bg_ref[...]
    col = lax.broadcasted_iota(jnp.int32, (TT, EP), 1)
    m = jnp.max(lg, axis=1, keepdims=True)
    el = jnp.exp(lg - m)
    s = jnp.sum(el, axis=1, keepdims=True)
    w = el / s
    w_ref[...] = w
    lp_ref[...] = lg - m - jnp.log(s)
    wm = jnp.where(col < E, w, -1.0)
    w1 = jnp.max(wm, axis=1, keepdims=True)
    i1 = jnp.min(jnp.where(wm == w1, col, EP), axis=1, keepdims=True)
    wm2 = jnp.where(col == i1, -1.0, wm)
    w2 = jnp.max(wm2, axis=1, keepdims=True)
    i2 = jnp.min(jnp.where(wm2 == w2, col, EP), axis=1, keepdims=True)
    den = w1 + w2 + 1e-9
    wn1 = w1 / den
    wn2 = w2 / den
    ti_ref[...] = jnp.where(col == 0, i1, jnp.where(col == 1, i2, 0))
    wb1_ref[...] = jnp.broadcast_to(wn1, (TT, EP))
    wb2_ref[...] = jnp.broadcast_to(wn2, (TT, EP))


def _gating(xf, wgp, bgp):
    grid = (T // TT,)
    f32 = jnp.float32
    return pl.pallas_call(
        _gating_body,
        grid=grid,
        in_specs=[
            pl.BlockSpec((TT, D), lambda i: (i, 0)),
            pl.BlockSpec((D, EP), lambda i: (0, 0)),
            pl.BlockSpec((1, EP), lambda i: (0, 0)),
        ],
        out_specs=[
            pl.BlockSpec((TT, EP), lambda i: (i, 0)),
            pl.BlockSpec((TT, EP), lambda i: (i, 0)),
            pl.BlockSpec((TT, EP), lambda i: (i, 0)),
            pl.BlockSpec((TT, EP), lambda i: (i, 0)),
            pl.BlockSpec((TT, EP), lambda i: (i, 0)),
        ],
        out_shape=[
            jax.ShapeDtypeStruct((T, EP), f32),
            jax.ShapeDtypeStruct((T, EP), f32),
            jax.ShapeDtypeStruct((T, EP), jnp.int32),
            jax.ShapeDtypeStruct((T, EP), f32),
            jax.ShapeDtypeStruct((T, EP), f32),
        ],
    )(xf, wgp, bgp)


def _sc_mesh():
    return plsc.VectorSubcoreMesh(core_axis_name="c", subcore_axis_name="s",
                                  num_cores=2)


def _worker_id():
    return lax.axis_index("s") * 2 + lax.axis_index("c")


def _dispatch_body(x_hbm, pe_hbm, po_hbm, xs_hbm, pe_v, po_v, xv, sem):
    base = _worker_id() * TPW
    pltpu.sync_copy(pe_hbm.at[pl.ds(base, TPW)], pe_v)
    pltpu.sync_copy(po_hbm.at[pl.ds(base, TPW)], po_v)
    pltpu.sync_copy(x_hbm.at[pl.ds(base, TPW)], xv)
    c0 = pltpu.async_copy(xv, xs_hbm.at[pe_v], sem)
    c1 = pltpu.async_copy(xv, xs_hbm.at[po_v], sem)
    c0.wait()
    c1.wait()


def _dispatch(xf, pe, po):
    return pl.kernel(
        _dispatch_body,
        out_type=jax.ShapeDtypeStruct((TK, D), jnp.float32),
        mesh=_sc_mesh(),
        scratch_types=[
            pltpu.VMEM((TPW,), jnp.int32),
            pltpu.VMEM((TPW,), jnp.int32),
            pltpu.VMEM((TPW, D), jnp.float32),
            pltpu.SemaphoreType.DMA,
        ],
    )(xf, pe, po)


def _mlp_body(off, xs_ref, w1_ref, b1_ref, w2_ref, b2_ref, out_ref):
    e = pl.program_id(0)
    lo = off[e]
    hi = off[e + 1]

    @pl.when(e == 0)
    def _():
        out_ref[...] = jnp.zeros_like(out_ref)

    t0 = lo // TM
    nt = jnp.where(hi > lo, (hi - 1) // TM - t0 + 1, 0)

    def tile(j, _):
        t = t0 + j
        r = pl.ds(t * TM, TM)
        h = jnp.dot(xs_ref[r, :], w1_ref[0],
                    preferred_element_type=jnp.float32)
        h = h + b1_ref[0]
        h = 0.5 * h * (1.0 + lax.erf(h * 0.7071067811865476))
        o = jnp.dot(h, w2_ref[0], preferred_element_type=jnp.float32)
        o = o + b2_ref[0]
        rows = t * TM + lax.broadcasted_iota(jnp.int32, (TM, 1), 0)
        keep = (rows >= lo) & (rows < hi)
        o = jnp.where(keep, o, 0.0)
        out_ref[r, :] += o
        return 0

    lax.fori_loop(0, nt, tile, 0)


def _grouped_mlp(xs, W1, b1, W2, b2, off9):
    grid_spec = pltpu.PrefetchScalarGridSpec(
        num_scalar_prefetch=1,
        grid=(E,),
        in_specs=[
            pl.BlockSpec((TK, D), lambda e, off: (0, 0)),
            pl.BlockSpec((1, D, H), lambda e, off: (e, 0, 0)),
            pl.BlockSpec((1, 1, H), lambda e, off: (e, 0, 0)),
            pl.BlockSpec((1, H, D), lambda e, off: (e, 0, 0)),
            pl.BlockSpec((1, 1, D), lambda e, off: (e, 0, 0)),
        ],
        out_specs=pl.BlockSpec((TK, D), lambda e, off: (0, 0)),
    )
    return pl.pallas_call(
        _mlp_body,
        grid_spec=grid_spec,
        out_shape=jax.ShapeDtypeStruct((TK, D), jnp.float32),
    )(off9, xs, W1,
      b1.reshape(E, 1, H), W2, b2.reshape(E, 1, D))


def _combine_body(ys_hbm, pe_hbm, po_hbm, wb1_hbm, wb2_hbm, y_hbm,
                  pe_v, po_v, w1_v, w2_v, av, bv, sem0, sem1):
    base = _worker_id() * TPW
    h2 = TPW // 2
    pltpu.sync_copy(pe_hbm.at[pl.ds(base, TPW)], pe_v)
    pltpu.sync_copy(po_hbm.at[pl.ds(base, TPW)], po_v)
    g0 = pltpu.async_copy(ys_hbm.at[pe_v.at[pl.ds(0, h2)]],
                          av.at[pl.ds(0, h2)], sem0)
    g1 = pltpu.async_copy(ys_hbm.at[po_v.at[pl.ds(0, h2)]],
                          bv.at[pl.ds(0, h2)], sem0)
    g2 = pltpu.async_copy(ys_hbm.at[pe_v.at[pl.ds(h2, h2)]],
                          av.at[pl.ds(h2, h2)], sem1)
    g3 = pltpu.async_copy(ys_hbm.at[po_v.at[pl.ds(h2, h2)]],
                          bv.at[pl.ds(h2, h2)], sem1)
    pltpu.sync_copy(wb1_hbm.at[pl.ds(base, TPW)], w1_v)
    pltpu.sync_copy(wb2_hbm.at[pl.ds(base, TPW)], w2_v)

    def half(off):
        @plsc.parallel_loop(0, h2 * (D // WL), unroll=8)
        def _fma(i):
            t = off + (i & (h2 - 1))
            j = lax.shift_right_logical(i, 5)
            sl = pl.ds(j * WL, WL)
            wsl = pl.ds(0, WL)
            av[t, sl] = av[t, sl] * w1_v[t, wsl] + bv[t, sl] * w2_v[t, wsl]

    g0.wait()
    g1.wait()
    half(0)
    g2.wait()
    g3.wait()
    half(h2)
    pltpu.sync_copy(av, y_hbm.at[pl.ds(base, TPW)])


def _combine(ys, pe, po, wb1, wb2):
    return pl.kernel(
        _combine_body,
        out_type=jax.ShapeDtypeStruct((T, D), jnp.float32),
        mesh=_sc_mesh(),
        scratch_types=[
            pltpu.VMEM((TPW,), jnp.int32),
            pltpu.VMEM((TPW,), jnp.int32),
            pltpu.VMEM((TPW, EP), jnp.float32),
            pltpu.VMEM((TPW, EP), jnp.float32),
            pltpu.VMEM((TPW, D), jnp.float32),
            pltpu.VMEM((TPW, D), jnp.float32),
            pltpu.SemaphoreType.DMA,
            pltpu.SemaphoreType.DMA,
        ],
    )(ys, pe, po, wb1, wb2)


def _schedule(ti):
    ex = ti.reshape(-1)
    onehot = (ex[:, None] == jnp.arange(E)[None, :]).astype(jnp.int32)
    counts = jnp.sum(onehot, axis=0)
    off9 = jnp.concatenate([jnp.zeros((1,), jnp.int32),
                            jnp.cumsum(counts).astype(jnp.int32)])
    rank = jnp.cumsum(onehot, axis=0) - onehot
    rank_p = jnp.take_along_axis(rank, ex[:, None], axis=1)[:, 0]
    pos = off9[ex] + rank_p
    pe = pos[0::2].astype(jnp.int32)
    po = pos[1::2].astype(jnp.int32)
    return pe, po, off9


def kernel(x, Wg, bg, W1, b1, W2, b2):
    b, s, d = x.shape
    xf = x.reshape(T, D)
    wgp = jnp.pad(Wg, ((0, 0), (0, EP - E)))
    bgp = jnp.pad(bg, (0, EP - E), constant_values=-1e30).reshape(1, EP)

    w_pad, lp_pad, ti_pad, wb1, wb2 = _gating(xf, wgp, bgp)
    weights = w_pad[:, :E]
    log_probs = lp_pad[:, :E]
    ti = ti_pad[:, :KTOP]

    pe, po, off9 = _schedule(ti)
    xs = _dispatch(xf, pe, po)
    ys = _grouped_mlp(xs, W1, b1, W2, b2, off9)
    y = _combine(ys, pe, po, wb1, wb2)

    return (y.reshape(b, s, d), log_probs.reshape(b, s, E),
            weights.reshape(b, s, E), ti.reshape(b, s, KTOP))
